# baseline (speedup 1.0000x reference)
import functools

import jax
import jax.numpy as jnp
from jax import lax
from jax.experimental import pallas as pl
from jax.experimental.pallas import tpu as pltpu
from jax.experimental.pallas import tpu_sc as plsc

N_FLAT = 2048
D_EMB = 64
N_CODES = 8192
K_BLK = 1024

NC = 2
NS = 16
NW = NC * NS
BPW = N_FLAT // NW


def _argmin_body(flat_ref, embt_ref, minval_ref, minidx_ref):
    j = pl.program_id(0)
    f = flat_ref[...]
    et = embt_ref[...]
    scores = jnp.dot(f, et, preferred_element_type=jnp.float32) * (-2.0)
    scores = scores + jnp.sum(et * et, axis=0, keepdims=True)
    local_min = jnp.min(scores, axis=1, keepdims=True)
    ids = lax.broadcasted_iota(jnp.int32, scores.shape, 1)
    cand = jnp.where(scores <= local_min, ids, jnp.int32(2 ** 30))
    local_arg = jnp.min(cand, axis=1, keepdims=True) + j * K_BLK

    @pl.when(j == 0)
    def _():
        minval_ref[...] = local_min
        minidx_ref[...] = local_arg

    @pl.when(j > 0)
    def _():
        better = local_min < minval_ref[...]
        minval_ref[...] = jnp.where(better, local_min, minval_ref[...])
        minidx_ref[...] = jnp.where(better, local_arg, minidx_ref[...])


def _vq_argmin(flat, embt):
    out = pl.pallas_call(
        _argmin_body,
        grid=(N_CODES // K_BLK,),
        in_specs=[
            pl.BlockSpec((N_FLAT, D_EMB), lambda j: (0, 0)),
            pl.BlockSpec((D_EMB, K_BLK), lambda j: (0, j)),
        ],
        out_specs=[
            pl.BlockSpec((N_FLAT, 1), lambda j: (0, 0)),
            pl.BlockSpec((N_FLAT, 1), lambda j: (0, 0)),
        ],
        out_shape=[
            jax.ShapeDtypeStruct((N_FLAT, 1), jnp.float32),
            jax.ShapeDtypeStruct((N_FLAT, 1), jnp.int32),
        ],
    )(flat, embt)
    return out[1]


def _sc_body(emb_hbm, idx_hbm, quant_hbm, idx_v, rows_v, sem):
    c = lax.axis_index("c")
    s = lax.axis_index("s")
    wid = s * NC + c
    base = wid * BPW

    pltpu.sync_copy(idx_hbm.at[pl.ds(base, BPW)], idx_v)
    pltpu.async_copy(emb_hbm.at[idx_v], rows_v, sem).wait()
    pltpu.sync_copy(rows_v, quant_hbm.at[pl.ds(base, BPW)])


@functools.lru_cache(maxsize=1)
def _sc_gather_fn():
    mesh = plsc.VectorSubcoreMesh(core_axis_name="c", subcore_axis_name="s")
    return pl.kernel(
        _sc_body,
        mesh=mesh,
        compiler_params=pltpu.CompilerParams(use_tc_tiling_on_sc=False),
        out_type=jax.ShapeDtypeStruct((N_FLAT, D_EMB), jnp.float32),
        scratch_types=[
            pltpu.VMEM((BPW,), jnp.int32),
            pltpu.VMEM((BPW, D_EMB), jnp.float32),
            pltpu.SemaphoreType.DMA,
        ],
    )


def _hist_body(idx_ref, counts_ref):
    j = pl.program_id(0)
    ids = idx_ref[...]
    cols = lax.broadcasted_iota(jnp.int32, (N_FLAT, K_BLK), 1) + j * K_BLK
    onehot = (ids == cols).astype(jnp.float32)
    counts_ref[...] = jnp.sum(onehot, axis=0)[None, None, :]


def _vq_hist(idx2d):
    out = pl.pallas_call(
        _hist_body,
        grid=(N_CODES // K_BLK,),
        in_specs=[pl.BlockSpec((N_FLAT, 1), lambda j: (0, 0))],
        out_specs=pl.BlockSpec((1, 1, K_BLK), lambda j: (j, 0, 0)),
        out_shape=jax.ShapeDtypeStruct((N_CODES // K_BLK, 1, K_BLK), jnp.float32),
    )(idx2d)
    return out.reshape(N_CODES)


_S_OUT = 8192


def _decout_body(y_ref, w_ref, b_ref, out_ref):
    yv = y_ref[0]
    w = w_ref[...]
    out = lax.dot_general(w, yv, (((1,), (0,)), ((), ())),
                          preferred_element_type=jnp.float32)
    out_ref[0] = out + b_ref[...]


def _decout(y, w, b):
    n, ci, h, wd = y.shape
    co = w.shape[0]
    y3 = y.reshape(n, ci, h * wd)
    out = pl.pallas_call(
        _decout_body,
        grid=(n, (h * wd) // _S_OUT),
        in_specs=[
            pl.BlockSpec((1, ci, _S_OUT), lambda i, j: (i, 0, j)),
            pl.BlockSpec((co, ci), lambda i, j: (0, 0)),
            pl.BlockSpec((co, 1), lambda i, j: (0, 0)),
        ],
        out_specs=pl.BlockSpec((1, co, _S_OUT), lambda i, j: (i, 0, j)),
        out_shape=jax.ShapeDtypeStruct((n, co, h * wd), jnp.float32),
    )(y3, w.reshape(co, ci), b.reshape(co, 1))
    return out.reshape(n, co, h, wd)


def _conv2d(x, w, b, stride, pad):
    y = lax.conv_general_dilated(
        x, w, (stride, stride), ((pad, pad), (pad, pad)),
        dimension_numbers=('NCHW', 'OIHW', 'NCHW'))
    return y + b[None, :, None, None]


def _convT2d(x, w, b, stride=2, pad=1):
    k = w.shape[2]
    w2 = jnp.transpose(jnp.flip(w, (2, 3)), (1, 0, 2, 3))
    p = k - 1 - pad
    y = lax.conv_general_dilated(
        x, w2, (1, 1), ((p, p), (p, p)), lhs_dilation=(stride, stride),
        dimension_numbers=('NCHW', 'OIHW', 'NCHW'))
    return y + b[None, :, None, None]


def _convT_s1(x, w, b):
    ci, co = w.shape[0], w.shape[1]
    w2 = jnp.transpose(jnp.flip(w, (2, 3)), (1, 0, 2, 3))
    ky_map = {(0, -1): 0, (0, 0): 2, (1, 0): 1, (1, 1): 3}
    W3 = jnp.zeros((4 * co, ci, 3, 3), w.dtype)
    for r in (0, 1):
        for s in (0, 1):
            for dy in (-1, 0, 1):
                if (r, dy) not in ky_map:
                    continue
                for dx in (-1, 0, 1):
                    if (s, dx) not in ky_map:
                        continue
                    W3 = W3.at[(r * 2 + s) * co:(r * 2 + s + 1) * co, :,
                               dy + 1, dx + 1].set(w2[:, :, ky_map[(r, dy)],
                                                      ky_map[(s, dx)]])
    n, _, H, W = x.shape
    y4 = lax.conv_general_dilated(x, W3, (1, 1), ((1, 1), (1, 1)),
                                  dimension_numbers=('NCHW', 'OIHW', 'NCHW'))
    y4 = jax.nn.relu(y4 + jnp.tile(b, 4)[None, :, None, None])
    y4 = y4.reshape(n, 2, 2, co, H, W)
    y4 = jnp.transpose(y4, (0, 3, 4, 1, 5, 2))
    return y4.reshape(n, co, 2 * H, 2 * W)


def _resblock(x, P, pre):
    h = jax.nn.relu(_conv2d(x, P[pre + '_0_w'], P[pre + '_0_b'], 1, 1))
    h = jax.nn.relu(_conv2d(h, P[pre + '_1_w'], P[pre + '_1_b'], 1, 1))
    h = _conv2d(h, P[pre + '_2_w'], P[pre + '_2_b'], 1, 0)
    return h + x


def _conv2d_b(x, w, b, stride, pad, relu=True):
    y = lax.conv_general_dilated(
        x, w.astype(jnp.bfloat16), (stride, stride), ((pad, pad), (pad, pad)),
        dimension_numbers=('NCHW', 'OIHW', 'NCHW'),
        preferred_element_type=jnp.float32)
    y = y + b[None, :, None, None]
    if relu:
        y = jnp.maximum(y, 0.0)
    return y.astype(jnp.bfloat16)


def _convT2d_b(x, w, b):
    k = w.shape[2]
    w2 = jnp.transpose(jnp.flip(w, (2, 3)), (1, 0, 2, 3)).astype(jnp.bfloat16)
    p = k - 1 - 1
    y = lax.conv_general_dilated(
        x, w2, (1, 1), ((p, p), (p, p)), lhs_dilation=(2, 2),
        dimension_numbers=('NCHW', 'OIHW', 'NCHW'),
        preferred_element_type=jnp.float32)
    y = jnp.maximum(y + b[None, :, None, None], 0.0)
    return y.astype(jnp.bfloat16)


def _resblock_b(x, P, pre):
    h = _conv2d_b(x, P[pre + '_0_w'], P[pre + '_0_b'], 1, 1)
    h = _conv2d_b(h, P[pre + '_1_w'], P[pre + '_1_b'], 1, 1)
    h = _conv2d_b(h, P[pre + '_2_w'], P[pre + '_2_b'], 1, 0, relu=False)
    return h + x


def kernel(img, emb, enc0_w, enc0_b, enc1_w, enc1_b, enc2_w, enc2_b, enc3_w, enc3_b, encres0_0_w, encres0_0_b, encres0_1_w, encres0_1_b, encres0_2_w, encres0_2_b, encres1_0_w, encres1_0_b, encres1_1_w, encres1_1_b, encres1_2_w, encres1_2_b, encout_w, encout_b, dec0_w, dec0_b, decres0_0_w, decres0_0_b, decres0_1_w, decres0_1_b, decres0_2_w, decres0_2_b, decres1_0_w, decres1_0_b, decres1_1_w, decres1_1_b, decres1_2_w, decres1_2_b, dect0_w, dect0_b, dect1_w, dect1_b, dect2_w, dect2_b, dect3_w, dect3_b, decout_w, decout_b):
    P = dict(locals())
    img = P.pop('img')
    emb = P.pop('emb')

    x = jax.nn.relu(_conv2d(img, P['enc0_w'], P['enc0_b'], 2, 1))
    for n in (1, 2, 3):
        x = jax.nn.relu(_conv2d(x, P['enc%d_w' % n], P['enc%d_b' % n], 2, 1))
    for r in (0, 1):
        x = _resblock(x, P, 'encres%d' % r)
    z = _conv2d(x, P['encout_w'], P['encout_b'], 1, 0)
    zc = jnp.transpose(z, (0, 2, 3, 1))
    flat = zc.reshape(-1, zc.shape[-1])

    idx2d = _vq_argmin(flat, emb.T)
    quant_flat = _sc_gather_fn()(emb, idx2d[:, 0])
    counts = _vq_hist(idx2d)

    e_latent = jnp.mean((quant_flat - flat) ** 2)
    vq_loss = 0.25 * e_latent
    avg = counts / N_FLAT
    perplexity = jnp.exp(-jnp.sum(avg * jnp.log(avg + 1e-10)))

    quant = quant_flat.reshape(zc.shape)
    q = jnp.transpose(quant, (0, 3, 1, 2))
    y = jax.nn.relu(_conv2d(q, P['dec0_w'], P['dec0_b'], 1, 0))
    for r in (0, 1):
        y = _resblock(y, P, 'decres%d' % r)
    for n in (0, 1, 2, 3):
        y = _convT_s1(y, P['dect%d_w' % n], P['dect%d_b' % n])
    recon = _decout(y, P['decout_w'][:, :, 0, 0], P['decout_b'])
    recon_loss = jnp.mean((recon - img) ** 2)
    loss = recon_loss + vq_loss
    return (loss, recon, perplexity)


# speedup vs baseline: 1.2590x; 1.2590x over previous
import functools

import jax
import jax.numpy as jnp
from jax import lax
from jax.experimental import pallas as pl
from jax.experimental.pallas import tpu as pltpu
from jax.experimental.pallas import tpu_sc as plsc

N_FLAT = 2048
D_EMB = 64
N_CODES = 8192
K_BLK = 1024

NC = 2
NS = 16
NW = NC * NS
BPW = N_FLAT // NW


def _argmin_body(flat_ref, embt_ref, minval_ref, minidx_ref):
    j = pl.program_id(0)
    f = flat_ref[...]
    et = embt_ref[...]
    scores = jnp.dot(f, et, preferred_element_type=jnp.float32) * (-2.0)
    scores = scores + jnp.sum(et * et, axis=0, keepdims=True)
    local_min = jnp.min(scores, axis=1, keepdims=True)
    ids = lax.broadcasted_iota(jnp.int32, scores.shape, 1)
    cand = jnp.where(scores <= local_min, ids, jnp.int32(2 ** 30))
    local_arg = jnp.min(cand, axis=1, keepdims=True) + j * K_BLK

    @pl.when(j == 0)
    def _():
        minval_ref[...] = local_min
        minidx_ref[...] = local_arg

    @pl.when(j > 0)
    def _():
        better = local_min < minval_ref[...]
        minval_ref[...] = jnp.where(better, local_min, minval_ref[...])
        minidx_ref[...] = jnp.where(better, local_arg, minidx_ref[...])


def _vq_argmin(flat, embt):
    out = pl.pallas_call(
        _argmin_body,
        grid=(N_CODES // K_BLK,),
        in_specs=[
            pl.BlockSpec((N_FLAT, D_EMB), lambda j: (0, 0)),
            pl.BlockSpec((D_EMB, K_BLK), lambda j: (0, j)),
        ],
        out_specs=[
            pl.BlockSpec((N_FLAT, 1), lambda j: (0, 0)),
            pl.BlockSpec((N_FLAT, 1), lambda j: (0, 0)),
        ],
        out_shape=[
            jax.ShapeDtypeStruct((N_FLAT, 1), jnp.float32),
            jax.ShapeDtypeStruct((N_FLAT, 1), jnp.int32),
        ],
    )(flat, embt)
    return out[1]


def _sc_body(emb_hbm, idx_hbm, quant_hbm, idx_v, rows_v, sem):
    c = lax.axis_index("c")
    s = lax.axis_index("s")
    wid = s * NC + c
    base = wid * BPW

    pltpu.sync_copy(idx_hbm.at[pl.ds(base, BPW)], idx_v)
    pltpu.async_copy(emb_hbm.at[idx_v], rows_v, sem).wait()
    pltpu.sync_copy(rows_v, quant_hbm.at[pl.ds(base, BPW)])


@functools.lru_cache(maxsize=1)
def _sc_gather_fn():
    mesh = plsc.VectorSubcoreMesh(core_axis_name="c", subcore_axis_name="s")
    return pl.kernel(
        _sc_body,
        mesh=mesh,
        compiler_params=pltpu.CompilerParams(use_tc_tiling_on_sc=False),
        out_type=jax.ShapeDtypeStruct((N_FLAT, D_EMB), jnp.float32),
        scratch_types=[
            pltpu.VMEM((BPW,), jnp.int32),
            pltpu.VMEM((BPW, D_EMB), jnp.float32),
            pltpu.SemaphoreType.DMA,
        ],
    )


def _hist_body(idx_ref, counts_ref):
    j = pl.program_id(0)
    ids = idx_ref[...]
    cols = lax.broadcasted_iota(jnp.int32, (N_FLAT, K_BLK), 1) + j * K_BLK
    onehot = (ids == cols).astype(jnp.float32)
    counts_ref[...] = jnp.sum(onehot, axis=0)[None, None, :]


def _vq_hist(idx2d):
    out = pl.pallas_call(
        _hist_body,
        grid=(N_CODES // K_BLK,),
        in_specs=[pl.BlockSpec((N_FLAT, 1), lambda j: (0, 0))],
        out_specs=pl.BlockSpec((1, 1, K_BLK), lambda j: (j, 0, 0)),
        out_shape=jax.ShapeDtypeStruct((N_CODES // K_BLK, 1, K_BLK), jnp.float32),
    )(idx2d)
    return out.reshape(N_CODES)


_S_OUT = 8192


def _decout_body(y_ref, w_ref, b_ref, out_ref):
    yv = y_ref[0]
    w = w_ref[...]
    out = lax.dot_general(w, yv, (((1,), (0,)), ((), ())),
                          preferred_element_type=jnp.float32)
    out_ref[0] = out + b_ref[...]


def _decout(y, w, b):
    n, ci, h, wd = y.shape
    co = w.shape[0]
    y3 = y.reshape(n, ci, h * wd)
    out = pl.pallas_call(
        _decout_body,
        grid=(n, (h * wd) // _S_OUT),
        in_specs=[
            pl.BlockSpec((1, ci, _S_OUT), lambda i, j: (i, 0, j)),
            pl.BlockSpec((co, ci), lambda i, j: (0, 0)),
            pl.BlockSpec((co, 1), lambda i, j: (0, 0)),
        ],
        out_specs=pl.BlockSpec((1, co, _S_OUT), lambda i, j: (i, 0, j)),
        out_shape=jax.ShapeDtypeStruct((n, co, h * wd), jnp.float32),
    )(y3, w.reshape(co, ci), b.reshape(co, 1))
    return out.reshape(n, co, h, wd)


def _conv2d(x, w, b, stride, pad):
    y = lax.conv_general_dilated(
        x, w, (stride, stride), ((pad, pad), (pad, pad)),
        dimension_numbers=('NCHW', 'OIHW', 'NCHW'))
    return y + b[None, :, None, None]


def _convT2d(x, w, b, stride=2, pad=1):
    k = w.shape[2]
    w2 = jnp.transpose(jnp.flip(w, (2, 3)), (1, 0, 2, 3))
    p = k - 1 - pad
    y = lax.conv_general_dilated(
        x, w2, (1, 1), ((p, p), (p, p)), lhs_dilation=(stride, stride),
        dimension_numbers=('NCHW', 'OIHW', 'NCHW'))
    return y + b[None, :, None, None]


def _convT_s1(x, w, b):
    ci, co = w.shape[0], w.shape[1]
    w2 = jnp.transpose(jnp.flip(w, (2, 3)), (1, 0, 2, 3))
    ky_map = {(0, -1): 0, (0, 0): 2, (1, 0): 1, (1, 1): 3}
    W3 = jnp.zeros((4 * co, ci, 3, 3), w.dtype)
    for r in (0, 1):
        for s in (0, 1):
            for dy in (-1, 0, 1):
                if (r, dy) not in ky_map:
                    continue
                for dx in (-1, 0, 1):
                    if (s, dx) not in ky_map:
                        continue
                    W3 = W3.at[(r * 2 + s) * co:(r * 2 + s + 1) * co, :,
                               dy + 1, dx + 1].set(w2[:, :, ky_map[(r, dy)],
                                                      ky_map[(s, dx)]])
    n, _, H, W = x.shape
    y4 = lax.conv_general_dilated(x, W3, (1, 1), ((1, 1), (1, 1)),
                                  dimension_numbers=('NCHW', 'OIHW', 'NCHW'))
    return jax.nn.relu(y4 + jnp.tile(b, 4)[None, :, None, None])


def _depth2space(y4, co):
    n, _, H, W = y4.shape
    y4 = y4.reshape(n, 2, 2, co, H, W)
    y4 = jnp.transpose(y4, (0, 3, 4, 1, 5, 2))
    return y4.reshape(n, co, 2 * H, 2 * W)


def _resblock(x, P, pre):
    h = jax.nn.relu(_conv2d(x, P[pre + '_0_w'], P[pre + '_0_b'], 1, 1))
    h = jax.nn.relu(_conv2d(h, P[pre + '_1_w'], P[pre + '_1_b'], 1, 1))
    h = _conv2d(h, P[pre + '_2_w'], P[pre + '_2_b'], 1, 0)
    return h + x


def _conv2d_b(x, w, b, stride, pad, relu=True):
    y = lax.conv_general_dilated(
        x, w.astype(jnp.bfloat16), (stride, stride), ((pad, pad), (pad, pad)),
        dimension_numbers=('NCHW', 'OIHW', 'NCHW'),
        preferred_element_type=jnp.float32)
    y = y + b[None, :, None, None]
    if relu:
        y = jnp.maximum(y, 0.0)
    return y.astype(jnp.bfloat16)


def _convT2d_b(x, w, b):
    k = w.shape[2]
    w2 = jnp.transpose(jnp.flip(w, (2, 3)), (1, 0, 2, 3)).astype(jnp.bfloat16)
    p = k - 1 - 1
    y = lax.conv_general_dilated(
        x, w2, (1, 1), ((p, p), (p, p)), lhs_dilation=(2, 2),
        dimension_numbers=('NCHW', 'OIHW', 'NCHW'),
        preferred_element_type=jnp.float32)
    y = jnp.maximum(y + b[None, :, None, None], 0.0)
    return y.astype(jnp.bfloat16)


def _resblock_b(x, P, pre):
    h = _conv2d_b(x, P[pre + '_0_w'], P[pre + '_0_b'], 1, 1)
    h = _conv2d_b(h, P[pre + '_1_w'], P[pre + '_1_b'], 1, 1)
    h = _conv2d_b(h, P[pre + '_2_w'], P[pre + '_2_b'], 1, 0, relu=False)
    return h + x


def kernel(img, emb, enc0_w, enc0_b, enc1_w, enc1_b, enc2_w, enc2_b, enc3_w, enc3_b, encres0_0_w, encres0_0_b, encres0_1_w, encres0_1_b, encres0_2_w, encres0_2_b, encres1_0_w, encres1_0_b, encres1_1_w, encres1_1_b, encres1_2_w, encres1_2_b, encout_w, encout_b, dec0_w, dec0_b, decres0_0_w, decres0_0_b, decres0_1_w, decres0_1_b, decres0_2_w, decres0_2_b, decres1_0_w, decres1_0_b, decres1_1_w, decres1_1_b, decres1_2_w, decres1_2_b, dect0_w, dect0_b, dect1_w, dect1_b, dect2_w, dect2_b, dect3_w, dect3_b, decout_w, decout_b):
    P = dict(locals())
    img = P.pop('img')
    emb = P.pop('emb')

    x = jax.nn.relu(_conv2d(img, P['enc0_w'], P['enc0_b'], 2, 1))
    for n in (1, 2, 3):
        x = jax.nn.relu(_conv2d(x, P['enc%d_w' % n], P['enc%d_b' % n], 2, 1))
    for r in (0, 1):
        x = _resblock(x, P, 'encres%d' % r)
    z = _conv2d(x, P['encout_w'], P['encout_b'], 1, 0)
    zc = jnp.transpose(z, (0, 2, 3, 1))
    flat = zc.reshape(-1, zc.shape[-1])

    idx2d = _vq_argmin(flat, emb.T)
    quant_flat = _sc_gather_fn()(emb, idx2d[:, 0])
    counts = _vq_hist(idx2d)

    e_latent = jnp.mean((quant_flat - flat) ** 2)
    vq_loss = 0.25 * e_latent
    avg = counts / N_FLAT
    perplexity = jnp.exp(-jnp.sum(avg * jnp.log(avg + 1e-10)))

    quant = quant_flat.reshape(zc.shape)
    q = jnp.transpose(quant, (0, 3, 1, 2))
    y = jax.nn.relu(_conv2d(q, P['dec0_w'], P['dec0_b'], 1, 0))
    for r in (0, 1):
        y = _resblock(y, P, 'decres%d' % r)
    for n in (0, 1, 2):
        y = jax.nn.relu(_convT2d(y, P['dect%d_w' % n], P['dect%d_b' % n]))
    y4 = _convT_s1(y, P['dect3_w'], P['dect3_b'])
    wo = P['decout_w'][:, :, 0, 0]
    w12 = jnp.zeros((12, 64), jnp.float32)
    for rs in range(4):
        w12 = w12.at[rs * 3:(rs + 1) * 3, rs * 16:(rs + 1) * 16].set(wo)
    b12 = jnp.tile(P['decout_b'], 4)
    recon12 = _decout(y4, w12, b12)
    recon = _depth2space(recon12, 3)
    recon_loss = jnp.mean((recon - img) ** 2)
    loss = recon_loss + vq_loss
    return (loss, recon, perplexity)


# speedup vs baseline: 1.2644x; 1.0043x over previous
import functools

import jax
import jax.numpy as jnp
from jax import lax
from jax.experimental import pallas as pl
from jax.experimental.pallas import tpu as pltpu
from jax.experimental.pallas import tpu_sc as plsc

N_FLAT = 2048
D_EMB = 64
N_CODES = 8192
K_BLK = 2048

NC = 2
NS = 16
NW = NC * NS
BPW = N_FLAT // NW


def _argmin_body(flat_ref, embt_ref, minval_ref, minidx_ref):
    j = pl.program_id(0)
    f = flat_ref[...]
    et = embt_ref[...]
    scores = jnp.dot(f, et, preferred_element_type=jnp.float32) * (-2.0)
    scores = scores + jnp.sum(et * et, axis=0, keepdims=True)
    local_min = jnp.min(scores, axis=1, keepdims=True)
    ids = lax.broadcasted_iota(jnp.int32, scores.shape, 1)
    cand = jnp.where(scores <= local_min, ids, jnp.int32(2 ** 30))
    local_arg = jnp.min(cand, axis=1, keepdims=True) + j * K_BLK

    @pl.when(j == 0)
    def _():
        minval_ref[...] = local_min
        minidx_ref[...] = local_arg

    @pl.when(j > 0)
    def _():
        better = local_min < minval_ref[...]
        minval_ref[...] = jnp.where(better, local_min, minval_ref[...])
        minidx_ref[...] = jnp.where(better, local_arg, minidx_ref[...])


def _vq_argmin(flat, embt):
    out = pl.pallas_call(
        _argmin_body,
        grid=(N_CODES // K_BLK,),
        in_specs=[
            pl.BlockSpec((N_FLAT, D_EMB), lambda j: (0, 0)),
            pl.BlockSpec((D_EMB, K_BLK), lambda j: (0, j)),
        ],
        out_specs=[
            pl.BlockSpec((N_FLAT, 1), lambda j: (0, 0)),
            pl.BlockSpec((N_FLAT, 1), lambda j: (0, 0)),
        ],
        out_shape=[
            jax.ShapeDtypeStruct((N_FLAT, 1), jnp.float32),
            jax.ShapeDtypeStruct((N_FLAT, 1), jnp.int32),
        ],
    )(flat, embt)
    return out[1]


def _sc_body(emb_hbm, idx_hbm, quant_hbm, idx_v, rows_v, sem):
    c = lax.axis_index("c")
    s = lax.axis_index("s")
    wid = s * NC + c
    base = wid * BPW

    pltpu.sync_copy(idx_hbm.at[pl.ds(base, BPW)], idx_v)
    pltpu.async_copy(emb_hbm.at[idx_v], rows_v, sem).wait()
    pltpu.sync_copy(rows_v, quant_hbm.at[pl.ds(base, BPW)])


@functools.lru_cache(maxsize=1)
def _sc_gather_fn():
    mesh = plsc.VectorSubcoreMesh(core_axis_name="c", subcore_axis_name="s")
    return pl.kernel(
        _sc_body,
        mesh=mesh,
        compiler_params=pltpu.CompilerParams(use_tc_tiling_on_sc=False),
        out_type=jax.ShapeDtypeStruct((N_FLAT, D_EMB), jnp.float32),
        scratch_types=[
            pltpu.VMEM((BPW,), jnp.int32),
            pltpu.VMEM((BPW, D_EMB), jnp.float32),
            pltpu.SemaphoreType.DMA,
        ],
    )


def _hist_body(idx_ref, counts_ref):
    j = pl.program_id(0)
    ids = idx_ref[...]
    cols = lax.broadcasted_iota(jnp.int32, (N_FLAT, K_BLK), 1) + j * K_BLK
    onehot = (ids == cols).astype(jnp.float32)
    counts_ref[...] = jnp.sum(onehot, axis=0)[None, None, :]


def _vq_hist(idx2d):
    out = pl.pallas_call(
        _hist_body,
        grid=(N_CODES // K_BLK,),
        in_specs=[pl.BlockSpec((N_FLAT, 1), lambda j: (0, 0))],
        out_specs=pl.BlockSpec((1, 1, K_BLK), lambda j: (j, 0, 0)),
        out_shape=jax.ShapeDtypeStruct((N_CODES // K_BLK, 1, K_BLK), jnp.float32),
    )(idx2d)
    return out.reshape(N_CODES)


_S_OUT = 8192


def _decout_body(y_ref, w_ref, b_ref, out_ref):
    yv = y_ref[0]
    w = w_ref[...]
    out = lax.dot_general(w, yv, (((1,), (0,)), ((), ())),
                          preferred_element_type=jnp.float32)
    out_ref[0] = out + b_ref[...]


def _decout(y, w, b):
    n, ci, h, wd = y.shape
    co = w.shape[0]
    y3 = y.reshape(n, ci, h * wd)
    out = pl.pallas_call(
        _decout_body,
        grid=(n, (h * wd) // _S_OUT),
        in_specs=[
            pl.BlockSpec((1, ci, _S_OUT), lambda i, j: (i, 0, j)),
            pl.BlockSpec((co, ci), lambda i, j: (0, 0)),
            pl.BlockSpec((co, 1), lambda i, j: (0, 0)),
        ],
        out_specs=pl.BlockSpec((1, co, _S_OUT), lambda i, j: (i, 0, j)),
        out_shape=jax.ShapeDtypeStruct((n, co, h * wd), jnp.float32),
    )(y3, w.reshape(co, ci), b.reshape(co, 1))
    return out.reshape(n, co, h, wd)


def _conv2d(x, w, b, stride, pad):
    y = lax.conv_general_dilated(
        x, w, (stride, stride), ((pad, pad), (pad, pad)),
        dimension_numbers=('NCHW', 'OIHW', 'NCHW'))
    return y + b[None, :, None, None]


def _convT2d(x, w, b, stride=2, pad=1):
    k = w.shape[2]
    w2 = jnp.transpose(jnp.flip(w, (2, 3)), (1, 0, 2, 3))
    p = k - 1 - pad
    y = lax.conv_general_dilated(
        x, w2, (1, 1), ((p, p), (p, p)), lhs_dilation=(stride, stride),
        dimension_numbers=('NCHW', 'OIHW', 'NCHW'))
    return y + b[None, :, None, None]


def _convT_s1(x, w, b):
    ci, co = w.shape[0], w.shape[1]
    w2 = jnp.transpose(jnp.flip(w, (2, 3)), (1, 0, 2, 3))
    ky_map = {(0, -1): 0, (0, 0): 2, (1, 0): 1, (1, 1): 3}
    W3 = jnp.zeros((4 * co, ci, 3, 3), w.dtype)
    for r in (0, 1):
        for s in (0, 1):
            for dy in (-1, 0, 1):
                if (r, dy) not in ky_map:
                    continue
                for dx in (-1, 0, 1):
                    if (s, dx) not in ky_map:
                        continue
                    W3 = W3.at[(r * 2 + s) * co:(r * 2 + s + 1) * co, :,
                               dy + 1, dx + 1].set(w2[:, :, ky_map[(r, dy)],
                                                      ky_map[(s, dx)]])
    n, _, H, W = x.shape
    y4 = lax.conv_general_dilated(x, W3, (1, 1), ((1, 1), (1, 1)),
                                  dimension_numbers=('NCHW', 'OIHW', 'NCHW'))
    return jax.nn.relu(y4 + jnp.tile(b, 4)[None, :, None, None])


def _depth2space(y4, co):
    n, _, H, W = y4.shape
    y4 = y4.reshape(n, 2, 2, co, H, W)
    y4 = jnp.transpose(y4, (0, 3, 4, 1, 5, 2))
    return y4.reshape(n, co, 2 * H, 2 * W)


def _resblock(x, P, pre):
    h = jax.nn.relu(_conv2d(x, P[pre + '_0_w'], P[pre + '_0_b'], 1, 1))
    h = jax.nn.relu(_conv2d(h, P[pre + '_1_w'], P[pre + '_1_b'], 1, 1))
    h = _conv2d(h, P[pre + '_2_w'], P[pre + '_2_b'], 1, 0)
    return h + x


def kernel(img, emb, enc0_w, enc0_b, enc1_w, enc1_b, enc2_w, enc2_b, enc3_w, enc3_b, encres0_0_w, encres0_0_b, encres0_1_w, encres0_1_b, encres0_2_w, encres0_2_b, encres1_0_w, encres1_0_b, encres1_1_w, encres1_1_b, encres1_2_w, encres1_2_b, encout_w, encout_b, dec0_w, dec0_b, decres0_0_w, decres0_0_b, decres0_1_w, decres0_1_b, decres0_2_w, decres0_2_b, decres1_0_w, decres1_0_b, decres1_1_w, decres1_1_b, decres1_2_w, decres1_2_b, dect0_w, dect0_b, dect1_w, dect1_b, dect2_w, dect2_b, dect3_w, dect3_b, decout_w, decout_b):
    P = dict(locals())
    img = P.pop('img')
    emb = P.pop('emb')

    x = jax.nn.relu(_conv2d(img, P['enc0_w'], P['enc0_b'], 2, 1))
    for n in (1, 2, 3):
        x = jax.nn.relu(_conv2d(x, P['enc%d_w' % n], P['enc%d_b' % n], 2, 1))
    for r in (0, 1):
        x = _resblock(x, P, 'encres%d' % r)
    z = _conv2d(x, P['encout_w'], P['encout_b'], 1, 0)
    zc = jnp.transpose(z, (0, 2, 3, 1))
    flat = zc.reshape(-1, zc.shape[-1])

    idx2d = _vq_argmin(flat, emb.T)
    quant_flat = _sc_gather_fn()(emb, idx2d[:, 0])
    counts = _vq_hist(idx2d)

    e_latent = jnp.mean((quant_flat - flat) ** 2)
    vq_loss = 0.25 * e_latent
    avg = counts / N_FLAT
    perplexity = jnp.exp(-jnp.sum(avg * jnp.log(avg + 1e-10)))

    quant = quant_flat.reshape(zc.shape)
    q = jnp.transpose(quant, (0, 3, 1, 2))
    y = jax.nn.relu(_conv2d(q, P['dec0_w'], P['dec0_b'], 1, 0))
    for r in (0, 1):
        y = _resblock(y, P, 'decres%d' % r)
    for n in (0, 1, 2):
        y = jax.nn.relu(_convT2d(y, P['dect%d_w' % n], P['dect%d_b' % n]))
    y4 = _convT_s1(y, P['dect3_w'], P['dect3_b'])
    wo = P['decout_w'][:, :, 0, 0]
    w12 = jnp.zeros((12, 64), jnp.float32)
    for rs in range(4):
        w12 = w12.at[rs * 3:(rs + 1) * 3, rs * 16:(rs + 1) * 16].set(wo)
    b12 = jnp.tile(P['decout_b'], 4)
    recon12 = _decout(y4, w12, b12)
    recon = _depth2space(recon12, 3)
    recon_loss = jnp.mean((recon - img) ** 2)
    loss = recon_loss + vq_loss
    return (loss, recon, perplexity)
